# baseline (speedup 1.0000x reference)
import functools

import jax
import jax.numpy as jnp
from jax import lax
from jax.experimental import pallas as pl
from jax.experimental.pallas import tpu as pltpu
from jax.experimental.pallas import tpu_sc as plsc

NUM_NODES = 100000
MEM_DIM = 128
TIME_DIM = 16
EMB_DIM = 128
B = 16384

NC = 2
NS = 16
NW = NC * NS

NCHUNK = 4
CB = B // NCHUNK


def _sc_gather_body(rows_per_w, mem_hbm, src_hbm, pos_hbm, neg_hbm,
                    out_s, out_p, out_n,
                    idx_v, rows_v, sem):
    wid = lax.axis_index("s") * NC + lax.axis_index("c")
    base = wid * rows_per_w
    for idx_hbm, out_hbm in ((src_hbm, out_s), (pos_hbm, out_p), (neg_hbm, out_n)):
        pltpu.sync_copy(idx_hbm.at[pl.ds(base, rows_per_w)], idx_v)
        pltpu.async_copy(mem_hbm.at[idx_v], rows_v, sem).wait()
        pltpu.sync_copy(rows_v, out_hbm.at[pl.ds(base, rows_per_w)])


def _sc_gather(memory, src, pos_dst, neg_dst):
    nrows = src.shape[0]
    rows_per_w = nrows // NW
    mesh = plsc.VectorSubcoreMesh(core_axis_name="c", subcore_axis_name="s")
    row_t = jax.ShapeDtypeStruct((nrows, MEM_DIM), jnp.float32)
    fn = pl.kernel(
        functools.partial(_sc_gather_body, rows_per_w),
        out_type=(row_t, row_t, row_t),
        mesh=mesh,
        scratch_types=[
            pltpu.VMEM((rows_per_w,), jnp.int32),
            pltpu.VMEM((rows_per_w, MEM_DIM), jnp.float32),
            pltpu.SemaphoreType.DMA,
        ],
    )
    return fn(memory, src, pos_dst, neg_dst)


BLK = 2048


def _fast_cos(x):
    q = jnp.round(x * 0.6366197723675814)
    r = x - q * 1.5707963705062866
    r = r + q * 4.3711388e-08
    z = r * r
    sinp = r * (1.0 + z * (-0.16666667 + z * (0.0083333310
                + z * (-0.00019840874 + z * 2.7525562e-6))))
    cosp = 1.0 + z * (-0.5 + z * (0.041666638
                + z * (-0.0013888378 + z * 2.4760495e-5)))
    k = q.astype(jnp.int32) & 3
    val = jnp.where((k & 1) == 1, sinp, cosp)
    return jnp.where((k == 1) | (k == 2), -val, val)


def _tc_body(t_ref, gs_ref, gp_ref, gn_ref,
             w_time_ref, b_time_ref, W_emb_ref, b_emb_ref,
             W1_ref, b1_ref, W2_ref, b2_ref,
             pos_ref, neg_ref):
    wm = W_emb_ref[:MEM_DIM, :]
    wt = W_emb_ref[MEM_DIM:, :]
    w1a = W1_ref[:EMB_DIM, :]
    w1b = W1_ref[EMB_DIM:, :]
    f32 = jnp.float32

    A = jnp.dot(wm, w1a, preferred_element_type=f32)
    C = jnp.dot(wm, w1b, preferred_element_type=f32)
    w1s = w1a + w1b
    wt2 = jnp.dot(wt, w1s, preferred_element_type=f32)
    b_shared = (jnp.dot(b_emb_ref[...].reshape(1, EMB_DIM), w1s,
                        preferred_element_type=f32)
                + b1_ref[...].reshape(1, EMB_DIM))

    t_enc = _fast_cos(t_ref[...][:, None] * w_time_ref[...] + b_time_ref[...])
    U = jnp.dot(t_enc, wt2, preferred_element_type=f32) + b_shared

    gsA = jnp.dot(gs_ref[...], A, preferred_element_type=f32)
    h_pos = jax.nn.relu(gsA + jnp.dot(gp_ref[...], C, preferred_element_type=f32) + U)
    h_neg = jax.nn.relu(gsA + jnp.dot(gn_ref[...], C, preferred_element_type=f32) + U)

    w2 = W2_ref[...]
    b2 = b2_ref[...]
    pos_ref[...] = jnp.dot(h_pos, w2, preferred_element_type=f32) + b2
    neg_ref[...] = jnp.dot(h_neg, w2, preferred_element_type=f32) + b2


def _tc_dense(t, g_src, g_pos, g_neg, w_time, b_time, W_emb, b_emb, W1, b1, W2, b2):
    nrows = t.shape[0]
    n_blk = nrows // BLK
    full = lambda s: pl.BlockSpec(s, lambda i: (0,) * len(s))
    grid_spec = pltpu.PrefetchScalarGridSpec(
        num_scalar_prefetch=0,
        grid=(n_blk,),
        in_specs=[
            pl.BlockSpec((BLK,), lambda i: (i,)),
            pl.BlockSpec((BLK, MEM_DIM), lambda i: (i, 0)),
            pl.BlockSpec((BLK, MEM_DIM), lambda i: (i, 0)),
            pl.BlockSpec((BLK, MEM_DIM), lambda i: (i, 0)),
            full((1, TIME_DIM)),
            full((TIME_DIM,)),
            full((MEM_DIM + TIME_DIM, EMB_DIM)),
            full((EMB_DIM,)),
            full((2 * EMB_DIM, EMB_DIM)),
            full((EMB_DIM,)),
            full((EMB_DIM, 1)),
            full((1,)),
        ],
        out_specs=[
            pl.BlockSpec((BLK, 1), lambda i: (i, 0)),
            pl.BlockSpec((BLK, 1), lambda i: (i, 0)),
        ],
    )
    return pl.pallas_call(
        _tc_body,
        grid_spec=grid_spec,
        out_shape=[
            jax.ShapeDtypeStruct((nrows, 1), jnp.float32),
            jax.ShapeDtypeStruct((nrows, 1), jnp.float32),
        ],
    )(t, g_src, g_pos, g_neg, w_time, b_time, W_emb, b_emb, W1, b1, W2, b2)


@jax.jit
def kernel(src, pos_dst, neg_dst, t, raw_msg, memory,
           w_time, b_time, W_emb, b_emb, W1, b1, W2, b2):
    del raw_msg
    pos_parts, neg_parts = [], []
    for ci in range(NCHUNK):
        sl = slice(ci * CB, (ci + 1) * CB)
        g_src, g_pos, g_neg = _sc_gather(memory, src[sl], pos_dst[sl], neg_dst[sl])
        p, n = _tc_dense(t[sl], g_src, g_pos, g_neg,
                         w_time, b_time, W_emb, b_emb, W1, b1, W2, b2)
        pos_parts.append(p)
        neg_parts.append(n)
    if NCHUNK == 1:
        return (pos_parts[0], neg_parts[0])
    return (jnp.concatenate(pos_parts), jnp.concatenate(neg_parts))


# speedup vs baseline: 1.1212x; 1.1212x over previous
import functools

import jax
import jax.numpy as jnp
from jax import lax
from jax.experimental import pallas as pl
from jax.experimental.pallas import tpu as pltpu
from jax.experimental.pallas import tpu_sc as plsc

NUM_NODES = 100000
MEM_DIM = 128
TIME_DIM = 16
EMB_DIM = 128
B = 16384

NC = 2
NS = 16
NW = NC * NS

NCHUNK = 1
CB = B // NCHUNK

GCH = 256


def _sc_gather_body(rows_per_w, mem_hbm, src_hbm, pos_hbm, neg_hbm,
                    out_s, out_p, out_n,
                    idx_v, rows0, rows1, g0, g1, s0, s1):
    wid = lax.axis_index("s") * NC + lax.axis_index("c")
    base = wid * rows_per_w
    nchips = rows_per_w // GCH
    idx_hbms = (src_hbm, pos_hbm, neg_hbm)
    out_hbms = (out_s, out_p, out_n)
    for a in range(3):
        pltpu.sync_copy(idx_hbms[a].at[pl.ds(base, rows_per_w)],
                        idx_v.at[pl.ds(a * rows_per_w, rows_per_w)])
    bufs = (rows0, rows1)
    gsems = (g0, g1)
    ssems = (s0, s1)
    nch = 3 * nchips
    gathers = [None] * nch
    stores = [None] * nch
    for c in range(nch):
        p = c % 2
        if c >= 2:
            stores[c - 2].wait()
        gathers[c] = pltpu.async_copy(
            mem_hbm.at[idx_v.at[pl.ds(c * GCH, GCH)]], bufs[p], gsems[p])
        if c >= 1:
            q = (c - 1) % 2
            gathers[c - 1].wait()
            a, h = divmod(c - 1, nchips)
            stores[c - 1] = pltpu.async_copy(
                bufs[q], out_hbms[a].at[pl.ds(base + h * GCH, GCH)], ssems[q])
    gathers[nch - 1].wait()
    a, h = divmod(nch - 1, nchips)
    stores[nch - 1] = pltpu.async_copy(
        bufs[(nch - 1) % 2], out_hbms[a].at[pl.ds(base + h * GCH, GCH)],
        ssems[(nch - 1) % 2])
    stores[nch - 2].wait()
    stores[nch - 1].wait()


def _sc_gather(memory, src, pos_dst, neg_dst):
    nrows = src.shape[0]
    rows_per_w = nrows // NW
    mesh = plsc.VectorSubcoreMesh(core_axis_name="c", subcore_axis_name="s")
    row_t = jax.ShapeDtypeStruct((nrows, MEM_DIM), jnp.float32)
    fn = pl.kernel(
        functools.partial(_sc_gather_body, rows_per_w),
        out_type=(row_t, row_t, row_t),
        mesh=mesh,
        scratch_types=[
            pltpu.VMEM((3 * rows_per_w,), jnp.int32),
            pltpu.VMEM((GCH, MEM_DIM), jnp.float32),
            pltpu.VMEM((GCH, MEM_DIM), jnp.float32),
            pltpu.SemaphoreType.DMA,
            pltpu.SemaphoreType.DMA,
            pltpu.SemaphoreType.DMA,
            pltpu.SemaphoreType.DMA,
        ],
    )
    return fn(memory, src, pos_dst, neg_dst)


BLK = 2048


def _fast_cos(x):
    q = jnp.round(x * 0.6366197723675814)
    r = x - q * 1.5707963705062866
    r = r + q * 4.3711388e-08
    z = r * r
    sinp = r * (1.0 + z * (-0.16666667 + z * (0.0083333310
                + z * (-0.00019840874 + z * 2.7525562e-6))))
    cosp = 1.0 + z * (-0.5 + z * (0.041666638
                + z * (-0.0013888378 + z * 2.4760495e-5)))
    k = q.astype(jnp.int32) & 3
    val = jnp.where((k & 1) == 1, sinp, cosp)
    return jnp.where((k == 1) | (k == 2), -val, val)


def _tc_body(t_ref, gs_ref, gp_ref, gn_ref,
             w_time_ref, b_time_ref, W_emb_ref, b_emb_ref,
             W1_ref, b1_ref, W2_ref, b2_ref,
             pos_ref, neg_ref):
    wm = W_emb_ref[:MEM_DIM, :]
    wt = W_emb_ref[MEM_DIM:, :]
    w1a = W1_ref[:EMB_DIM, :]
    w1b = W1_ref[EMB_DIM:, :]
    f32 = jnp.float32

    A = jnp.dot(wm, w1a, preferred_element_type=f32)
    C = jnp.dot(wm, w1b, preferred_element_type=f32)
    w1s = w1a + w1b
    wt2 = jnp.dot(wt, w1s, preferred_element_type=f32)
    b_shared = (jnp.dot(b_emb_ref[...].reshape(1, EMB_DIM), w1s,
                        preferred_element_type=f32)
                + b1_ref[...].reshape(1, EMB_DIM))

    t_enc = _fast_cos(t_ref[...][:, None] * w_time_ref[...] + b_time_ref[...])
    U = jnp.dot(t_enc, wt2, preferred_element_type=f32) + b_shared

    gsA = jnp.dot(gs_ref[...], A, preferred_element_type=f32)
    h_pos = jax.nn.relu(gsA + jnp.dot(gp_ref[...], C, preferred_element_type=f32) + U)
    h_neg = jax.nn.relu(gsA + jnp.dot(gn_ref[...], C, preferred_element_type=f32) + U)

    w2 = W2_ref[...]
    b2 = b2_ref[...]
    pos_ref[...] = jnp.dot(h_pos, w2, preferred_element_type=f32) + b2
    neg_ref[...] = jnp.dot(h_neg, w2, preferred_element_type=f32) + b2


def _tc_dense(t, g_src, g_pos, g_neg, w_time, b_time, W_emb, b_emb, W1, b1, W2, b2):
    nrows = t.shape[0]
    n_blk = nrows // BLK
    full = lambda s: pl.BlockSpec(s, lambda i: (0,) * len(s))
    grid_spec = pltpu.PrefetchScalarGridSpec(
        num_scalar_prefetch=0,
        grid=(n_blk,),
        in_specs=[
            pl.BlockSpec((BLK,), lambda i: (i,)),
            pl.BlockSpec((BLK, MEM_DIM), lambda i: (i, 0)),
            pl.BlockSpec((BLK, MEM_DIM), lambda i: (i, 0)),
            pl.BlockSpec((BLK, MEM_DIM), lambda i: (i, 0)),
            full((1, TIME_DIM)),
            full((TIME_DIM,)),
            full((MEM_DIM + TIME_DIM, EMB_DIM)),
            full((EMB_DIM,)),
            full((2 * EMB_DIM, EMB_DIM)),
            full((EMB_DIM,)),
            full((EMB_DIM, 1)),
            full((1,)),
        ],
        out_specs=[
            pl.BlockSpec((BLK, 1), lambda i: (i, 0)),
            pl.BlockSpec((BLK, 1), lambda i: (i, 0)),
        ],
    )
    return pl.pallas_call(
        _tc_body,
        grid_spec=grid_spec,
        out_shape=[
            jax.ShapeDtypeStruct((nrows, 1), jnp.float32),
            jax.ShapeDtypeStruct((nrows, 1), jnp.float32),
        ],
    )(t, g_src, g_pos, g_neg, w_time, b_time, W_emb, b_emb, W1, b1, W2, b2)


@jax.jit
def kernel(src, pos_dst, neg_dst, t, raw_msg, memory,
           w_time, b_time, W_emb, b_emb, W1, b1, W2, b2):
    del raw_msg
    pos_parts, neg_parts = [], []
    for ci in range(NCHUNK):
        sl = slice(ci * CB, (ci + 1) * CB)
        g_src, g_pos, g_neg = _sc_gather(memory, src[sl], pos_dst[sl], neg_dst[sl])
        p, n = _tc_dense(t[sl], g_src, g_pos, g_neg,
                         w_time, b_time, W_emb, b_emb, W1, b1, W2, b2)
        pos_parts.append(p)
        neg_parts.append(n)
    if NCHUNK == 1:
        return (pos_parts[0], neg_parts[0])
    return (jnp.concatenate(pos_parts), jnp.concatenate(neg_parts))
